# Initial kernel scaffold; baseline (speedup 1.0000x reference)
import jax
import jax.numpy as jnp
from jax.experimental import pallas as pl


def kernel(x, edge_index, batch, ptr, centrality, W_init, b_init, W0, b0, W1, b1, W2, b2, ln0_g, ln0_b, ln1_g, ln1_b, W_cls, b_cls):
    raise NotImplementedError("write your pallas kernel here")


# speedup vs baseline: 3.9125x; 3.9125x over previous
import functools

import jax
import jax.numpy as jnp
from jax import lax
from jax.experimental import pallas as pl
from jax.experimental.pallas import tpu as pltpu
from jax.experimental.pallas import tpu_sc as plsc

_N = 10000
_E = 320000
_H = 128
_G = 100
_NW = 32
_CK = 128
_CHUNKS = 80
_EP = _NW * _CHUNKS * _CK
_NP = 10240
_EB = 1024
_NB = _EP // _EB
_W = 256
_SENT = _NP + 64


def _gather_body(row_hbm, y_hbm, out_hbm, row_i, buf, gsem):
    cid = lax.axis_index("c")
    sid = lax.axis_index("s")
    wid = sid * 2 + cid

    def _body(j, _):
        pltpu.sync_copy(row_hbm.at[wid, j], row_i)
        pltpu.async_copy(y_hbm.at[row_i], buf, gsem).wait()
        pltpu.sync_copy(
            buf, out_hbm.at[pl.ds((wid * _CHUNKS + j) * _CK, _CK)])
        return 0

    lax.fori_loop(0, _CHUNKS, _body, 0)


@functools.lru_cache(maxsize=1)
def _sc_gather():
    mesh = plsc.VectorSubcoreMesh(core_axis_name="c", subcore_axis_name="s")
    return pl.kernel(
        _gather_body,
        out_type=jax.ShapeDtypeStruct((_EP, _H), jnp.float32),
        mesh=mesh,
        scratch_types=[
            pltpu.VMEM((_CK,), jnp.int32),
            pltpu.VMEM((_CK, _H), jnp.float32),
            pltpu.SemaphoreType.DMA,
        ],
    )


def _deg_body(nb_ref, colb_ref, deg_ref, dacc_ref):
    b = pl.program_id(0)

    @pl.when(b == 0)
    def _():
        dacc_ref[...] = jnp.zeros((_NP, _H), jnp.float32)

    nb = nb_ref[b]
    lid = colb_ref[0, 0:1, :] - nb
    rows = lax.broadcasted_iota(jnp.int32, (_W, _EB), 0)
    oh = jnp.where(rows == jnp.broadcast_to(lid, (_W, _EB)), 1.0, 0.0)
    dsum = jnp.sum(oh, axis=1, keepdims=True)
    dacc_ref[pl.ds(nb, _W), :] += jnp.broadcast_to(dsum, (_W, _H))

    @pl.when(b == pl.num_programs(0) - 1)
    def _():
        deg_ref[...] = dacc_ref[...]


def _agg_body(nb_ref, colb_ref, msg_ref, agg_ref, acc_ref):
    b = pl.program_id(0)

    @pl.when(b == 0)
    def _():
        acc_ref[...] = jnp.zeros((_NP, _H), jnp.float32)

    nb = nb_ref[b]
    lid = colb_ref[0, 0:1, :] - nb
    rows = lax.broadcasted_iota(jnp.int32, (_W, _EB), 0)
    oh = jnp.where(rows == jnp.broadcast_to(lid, (_W, _EB)), 1.0, 0.0)
    acc_ref[pl.ds(nb, _W), :] += jnp.dot(
        oh, msg_ref[...], preferred_element_type=jnp.float32)

    @pl.when(b == pl.num_programs(0) - 1)
    def _():
        agg_ref[...] = acc_ref[...]


_BLK = 2048


def _pre_body(x_ref, w_init_ref, b_init_ref, w0_ref, deg_ref,
              y0_ref, dinv_ref):
    dinv = jax.lax.rsqrt(deg_ref[...] + 1.0)
    dinv_ref[...] = dinv
    x1 = jnp.dot(x_ref[...], w_init_ref[...],
                 preferred_element_type=jnp.float32) + b_init_ref[0:1, :]
    y0_ref[...] = jnp.dot(x1, w0_ref[...],
                          preferred_element_type=jnp.float32) * dinv


def _mid_body(p_ref, y_ref, dinv_ref, b_ref, g_ref, bb_ref, w_ref, yn_ref):
    dinv = dinv_ref[...]
    h = (p_ref[...] + y_ref[...]) * dinv + b_ref[0:1, :]
    h = jnp.maximum(h, 0.0)
    m = jnp.mean(h, axis=-1, keepdims=True)
    c = h - m
    v = jnp.mean(c * c, axis=-1, keepdims=True)
    h = c * jax.lax.rsqrt(v + 1e-5) * g_ref[0:1, :] + bb_ref[0:1, :]
    yn_ref[...] = jnp.dot(h, w_ref[...],
                          preferred_element_type=jnp.float32) * dinv


_FBLK = 1000


def _final_body(p_ref, y_ref, dinv_ref, b_ref, batch_ref,
                wcls_ref, bcls_ref, emb_ref, logp_ref, pool_ref, cnt_ref):
    pid = pl.program_id(0)

    @pl.when(pid == 0)
    def _():
        pool_ref[...] = jnp.zeros((_H, _H), jnp.float32)
        cnt_ref[...] = jnp.zeros((_H, _H), jnp.float32)

    x3 = (p_ref[...] + y_ref[...]) * dinv_ref[...] + b_ref[0:1, :]
    gids = lax.broadcasted_iota(jnp.int32, (_H, _FBLK), 0)
    bb = jnp.broadcast_to(batch_ref[0, 0:1, :], (_H, _FBLK))
    oh = jnp.where(gids == bb, 1.0, 0.0)
    pool_ref[...] += jnp.dot(oh, x3, preferred_element_type=jnp.float32)
    cnt_ref[...] += jnp.broadcast_to(
        jnp.sum(oh, axis=1, keepdims=True), (_H, _H))

    @pl.when(pid == pl.num_programs(0) - 1)
    def _():
        mean = pool_ref[...] / jnp.maximum(cnt_ref[...], 1.0)
        logits = jnp.dot(mean, wcls_ref[...],
                         preferred_element_type=jnp.float32) + bcls_ref[0:1, :]
        emb_ref[...] = logits
        cmask = lax.broadcasted_iota(jnp.int32, (_H, _H), 1) < 10
        ml = jnp.where(cmask, logits, -1e30)
        mx = jnp.max(ml, axis=-1, keepdims=True)
        lse = mx + jnp.log(jnp.sum(jnp.exp(ml - mx), axis=-1, keepdims=True))
        logp_ref[...] = logits - lse


def _row_spec(blk):
    return pl.BlockSpec((blk, _H), lambda i: (i, 0))


def _full_spec(shape):
    return pl.BlockSpec(shape, lambda i: tuple(0 for _ in shape))


def _rep8(v):
    return jnp.tile(v[None, :], (8, 1))


def kernel(x, edge_index, batch, ptr, centrality, W_init, b_init, W0, b0,
           W1, b1, W2, b2, ln0_g, ln0_b, ln1_g, ln1_b, W_cls, b_cls):
    f32 = jnp.float32
    pad_e = _EP - _E
    rowp = jnp.concatenate(
        [edge_index[0].astype(jnp.int32), jnp.zeros((pad_e,), jnp.int32)])
    colp = jnp.concatenate(
        [edge_index[1].astype(jnp.int32),
         jnp.full((pad_e,), _SENT, jnp.int32)])
    order = jnp.argsort(colp)
    row_s = rowp[order]
    col_s = colp[order]
    row_s3 = row_s.reshape(_NW, _CHUNKS, _CK)
    col_s3 = col_s.reshape(_NB, 1, _EB)
    nb = jnp.minimum((col_s[::_EB] // 8) * 8, _NP - _W).astype(jnp.int32)

    xp = jnp.zeros((_NP, _H), f32).at[:_N].set(x)

    deg = pl.pallas_call(
        _deg_body,
        grid_spec=pltpu.PrefetchScalarGridSpec(
            num_scalar_prefetch=1,
            grid=(_NB,),
            in_specs=[pl.BlockSpec((1, 1, _EB), lambda b, nbr: (b, 0, 0))],
            out_specs=pl.BlockSpec((_NP, _H), lambda b, nbr: (0, 0)),
            scratch_shapes=[pltpu.VMEM((_NP, _H), f32)],
        ),
        out_shape=jax.ShapeDtypeStruct((_NP, _H), f32),
    )(nb, col_s3)

    y0, dinv = pl.pallas_call(
        _pre_body,
        grid=(_NP // _BLK,),
        in_specs=[
            _row_spec(_BLK), _full_spec((_H, _H)), _full_spec((8, _H)),
            _full_spec((_H, _H)), _row_spec(_BLK),
        ],
        out_specs=[_row_spec(_BLK), _row_spec(_BLK)],
        out_shape=[jax.ShapeDtypeStruct((_NP, _H), f32),
                   jax.ShapeDtypeStruct((_NP, _H), f32)],
    )(xp, W_init, _rep8(b_init), W0, deg)

    gather_k = _sc_gather()

    def agg_layer(y):
        msg = gather_k(row_s3, y)
        return pl.pallas_call(
            _agg_body,
            grid_spec=pltpu.PrefetchScalarGridSpec(
                num_scalar_prefetch=1,
                grid=(_NB,),
                in_specs=[
                    pl.BlockSpec((1, 1, _EB), lambda b, nbr: (b, 0, 0)),
                    pl.BlockSpec((_EB, _H), lambda b, nbr: (b, 0)),
                ],
                out_specs=pl.BlockSpec((_NP, _H), lambda b, nbr: (0, 0)),
                scratch_shapes=[pltpu.VMEM((_NP, _H), f32)],
            ),
            out_shape=jax.ShapeDtypeStruct((_NP, _H), f32),
        )(nb, col_s3, msg)

    def mid(p, y_prev, b_prev, g, bbeta, w_next):
        return pl.pallas_call(
            _mid_body,
            grid=(_NP // _BLK,),
            in_specs=[
                _row_spec(_BLK), _row_spec(_BLK), _row_spec(_BLK),
                _full_spec((8, _H)), _full_spec((8, _H)),
                _full_spec((8, _H)), _full_spec((_H, _H)),
            ],
            out_specs=_row_spec(_BLK),
            out_shape=jax.ShapeDtypeStruct((_NP, _H), f32),
        )(p, y_prev, dinv, _rep8(b_prev), _rep8(g), _rep8(bbeta), w_next)

    agg0 = agg_layer(y0)
    y1 = mid(agg0, y0, b0, ln0_g, ln0_b, W1)
    agg1 = agg_layer(y1)
    y2 = mid(agg1, y1, b1, ln1_g, ln1_b, W2)
    agg2 = agg_layer(y2)

    wcls_p = jnp.zeros((_H, _H), f32).at[:, :10].set(W_cls)
    bcls_p = jnp.zeros((_H,), f32).at[:10].set(b_cls)
    batch3 = batch.astype(jnp.int32).reshape(_N // _FBLK, 1, _FBLK)

    emb, logp = pl.pallas_call(
        _final_body,
        grid=(_N // _FBLK,),
        in_specs=[
            _row_spec(_FBLK), _row_spec(_FBLK), _row_spec(_FBLK),
            _full_spec((8, _H)),
            pl.BlockSpec((1, 1, _FBLK), lambda i: (i, 0, 0)),
            _full_spec((_H, _H)), _full_spec((8, _H)),
        ],
        out_specs=[_full_spec((_H, _H)), _full_spec((_H, _H))],
        out_shape=[jax.ShapeDtypeStruct((_H, _H), f32),
                   jax.ShapeDtypeStruct((_H, _H), f32)],
        scratch_shapes=[pltpu.VMEM((_H, _H), f32),
                        pltpu.VMEM((_H, _H), f32)],
    )(agg2, y2, dinv, _rep8(b2), batch3, wcls_p, _rep8(bcls_p))

    return (emb[:_G, :10], logp[:_G, :10])


# speedup vs baseline: 4.3408x; 1.1095x over previous
import functools

import jax
import jax.numpy as jnp
from jax import lax
from jax.experimental import pallas as pl
from jax.experimental.pallas import tpu as pltpu
from jax.experimental.pallas import tpu_sc as plsc

_N = 10000
_E = 320000
_H = 128
_G = 100
_NW = 32
_CK = 128
_CHUNKS = 80
_EP = _NW * _CHUNKS * _CK
_NP = 10240
_EB = 1024
_NB = _EP // _EB
_W = 256
_SENT = _NP + 64


def _gather_body(row_hbm, y_hbm, out_hbm, row_v, buf0, buf1,
                 g0, g1, w0, w1):
    cid = lax.axis_index("c")
    sid = lax.axis_index("s")
    wid = sid * 2 + cid
    pltpu.sync_copy(row_hbm.at[wid], row_v)

    def _obase(j):
        return out_hbm.at[pl.ds((wid * _CHUNKS + j) * _CK, _CK)]

    pltpu.async_copy(y_hbm.at[row_v.at[0]], buf0, g0)
    n_pairs = _CHUNKS // 2

    def _body(i, _):
        j0 = 2 * i

        @pl.when(i > 0)
        def _():
            pltpu.make_async_copy(buf1, _obase(j0 - 1), w1).wait()

        pltpu.async_copy(y_hbm.at[row_v.at[j0 + 1]], buf1, g1)
        pltpu.make_async_copy(y_hbm.at[row_v.at[j0]], buf0, g0).wait()
        pltpu.async_copy(buf0, _obase(j0), w0)
        pltpu.make_async_copy(y_hbm.at[row_v.at[j0 + 1]], buf1, g1).wait()
        pltpu.async_copy(buf1, _obase(j0 + 1), w1)

        @pl.when(i < n_pairs - 1)
        def _():
            pltpu.make_async_copy(buf0, _obase(j0), w0).wait()
            pltpu.async_copy(y_hbm.at[row_v.at[j0 + 2]], buf0, g0)

        return 0

    lax.fori_loop(0, n_pairs, _body, 0)
    pltpu.make_async_copy(buf0, _obase(_CHUNKS - 2), w0).wait()
    pltpu.make_async_copy(buf1, _obase(_CHUNKS - 1), w1).wait()


@functools.lru_cache(maxsize=1)
def _sc_gather():
    mesh = plsc.VectorSubcoreMesh(core_axis_name="c", subcore_axis_name="s")
    return pl.kernel(
        _gather_body,
        out_type=jax.ShapeDtypeStruct((_EP, _H), jnp.float32),
        mesh=mesh,
        scratch_types=[
            pltpu.VMEM((_CHUNKS, _CK), jnp.int32),
            pltpu.VMEM((_CK, _H), jnp.float32),
            pltpu.VMEM((_CK, _H), jnp.float32),
            pltpu.SemaphoreType.DMA,
            pltpu.SemaphoreType.DMA,
            pltpu.SemaphoreType.DMA,
            pltpu.SemaphoreType.DMA,
        ],
    )


def _deg_body(nb_ref, colb_ref, deg_ref, dacc_ref):
    b = pl.program_id(0)

    @pl.when(b == 0)
    def _():
        dacc_ref[...] = jnp.zeros((_NP, _H), jnp.float32)

    nb = nb_ref[b]
    lid = colb_ref[0, 0:1, :] - nb
    rows = lax.broadcasted_iota(jnp.int32, (_W, _EB), 0)
    oh = jnp.where(rows == jnp.broadcast_to(lid, (_W, _EB)), 1.0, 0.0)
    dsum = jnp.sum(oh, axis=1, keepdims=True)
    dacc_ref[pl.ds(nb, _W), :] += jnp.broadcast_to(dsum, (_W, _H))

    @pl.when(b == pl.num_programs(0) - 1)
    def _():
        deg_ref[...] = dacc_ref[...]


def _agg_body(nb_ref, colb_ref, msg_ref, agg_ref, acc_ref):
    b = pl.program_id(0)

    @pl.when(b == 0)
    def _():
        acc_ref[...] = jnp.zeros((_NP, _H), jnp.float32)

    nb = nb_ref[b]
    lid = colb_ref[0, 0:1, :] - nb
    rows = lax.broadcasted_iota(jnp.int32, (_W, _EB), 0)
    oh = jnp.where(rows == jnp.broadcast_to(lid, (_W, _EB)), 1.0, 0.0)
    acc_ref[pl.ds(nb, _W), :] += jnp.dot(
        oh, msg_ref[...], preferred_element_type=jnp.float32)

    @pl.when(b == pl.num_programs(0) - 1)
    def _():
        agg_ref[...] = acc_ref[...]


_BLK = 2048


def _pre_body(x_ref, w_init_ref, b_init_ref, w0_ref, deg_ref,
              y0_ref, dinv_ref):
    dinv = jax.lax.rsqrt(deg_ref[...] + 1.0)
    dinv_ref[...] = dinv
    x1 = jnp.dot(x_ref[...], w_init_ref[...],
                 preferred_element_type=jnp.float32) + b_init_ref[0:1, :]
    y0_ref[...] = jnp.dot(x1, w0_ref[...],
                          preferred_element_type=jnp.float32) * dinv


def _mid_body(p_ref, y_ref, dinv_ref, b_ref, g_ref, bb_ref, w_ref, yn_ref):
    dinv = dinv_ref[...]
    h = (p_ref[...] + y_ref[...]) * dinv + b_ref[0:1, :]
    h = jnp.maximum(h, 0.0)
    m = jnp.mean(h, axis=-1, keepdims=True)
    c = h - m
    v = jnp.mean(c * c, axis=-1, keepdims=True)
    h = c * jax.lax.rsqrt(v + 1e-5) * g_ref[0:1, :] + bb_ref[0:1, :]
    yn_ref[...] = jnp.dot(h, w_ref[...],
                          preferred_element_type=jnp.float32) * dinv


_FBLK = 1000


def _final_body(p_ref, y_ref, dinv_ref, b_ref, batch_ref,
                wcls_ref, bcls_ref, emb_ref, logp_ref, pool_ref, cnt_ref):
    pid = pl.program_id(0)

    @pl.when(pid == 0)
    def _():
        pool_ref[...] = jnp.zeros((_H, _H), jnp.float32)
        cnt_ref[...] = jnp.zeros((_H, _H), jnp.float32)

    x3 = (p_ref[...] + y_ref[...]) * dinv_ref[...] + b_ref[0:1, :]
    gids = lax.broadcasted_iota(jnp.int32, (_H, _FBLK), 0)
    bb = jnp.broadcast_to(batch_ref[0, 0:1, :], (_H, _FBLK))
    oh = jnp.where(gids == bb, 1.0, 0.0)
    pool_ref[...] += jnp.dot(oh, x3, preferred_element_type=jnp.float32)
    cnt_ref[...] += jnp.broadcast_to(
        jnp.sum(oh, axis=1, keepdims=True), (_H, _H))

    @pl.when(pid == pl.num_programs(0) - 1)
    def _():
        mean = pool_ref[...] / jnp.maximum(cnt_ref[...], 1.0)
        logits = jnp.dot(mean, wcls_ref[...],
                         preferred_element_type=jnp.float32) + bcls_ref[0:1, :]
        emb_ref[...] = logits
        cmask = lax.broadcasted_iota(jnp.int32, (_H, _H), 1) < 10
        ml = jnp.where(cmask, logits, -1e30)
        mx = jnp.max(ml, axis=-1, keepdims=True)
        lse = mx + jnp.log(jnp.sum(jnp.exp(ml - mx), axis=-1, keepdims=True))
        logp_ref[...] = logits - lse


def _row_spec(blk):
    return pl.BlockSpec((blk, _H), lambda i: (i, 0))


def _full_spec(shape):
    return pl.BlockSpec(shape, lambda i: tuple(0 for _ in shape))


def _rep8(v):
    return jnp.tile(v[None, :], (8, 1))


def kernel(x, edge_index, batch, ptr, centrality, W_init, b_init, W0, b0,
           W1, b1, W2, b2, ln0_g, ln0_b, ln1_g, ln1_b, W_cls, b_cls):
    f32 = jnp.float32
    pad_e = _EP - _E
    rowp = jnp.concatenate(
        [edge_index[0].astype(jnp.int32), jnp.zeros((pad_e,), jnp.int32)])
    colp = jnp.concatenate(
        [edge_index[1].astype(jnp.int32),
         jnp.full((pad_e,), _SENT, jnp.int32)])
    order = jnp.argsort(colp)
    row_s = rowp[order]
    col_s = colp[order]
    row_s3 = row_s.reshape(_NW, _CHUNKS, _CK)
    col_s3 = col_s.reshape(_NB, 1, _EB)
    nb = jnp.minimum((col_s[::_EB] // 8) * 8, _NP - _W).astype(jnp.int32)

    xp = jnp.zeros((_NP, _H), f32).at[:_N].set(x)

    deg = pl.pallas_call(
        _deg_body,
        grid_spec=pltpu.PrefetchScalarGridSpec(
            num_scalar_prefetch=1,
            grid=(_NB,),
            in_specs=[pl.BlockSpec((1, 1, _EB), lambda b, nbr: (b, 0, 0))],
            out_specs=pl.BlockSpec((_NP, _H), lambda b, nbr: (0, 0)),
            scratch_shapes=[pltpu.VMEM((_NP, _H), f32)],
        ),
        out_shape=jax.ShapeDtypeStruct((_NP, _H), f32),
    )(nb, col_s3)

    y0, dinv = pl.pallas_call(
        _pre_body,
        grid=(_NP // _BLK,),
        in_specs=[
            _row_spec(_BLK), _full_spec((_H, _H)), _full_spec((8, _H)),
            _full_spec((_H, _H)), _row_spec(_BLK),
        ],
        out_specs=[_row_spec(_BLK), _row_spec(_BLK)],
        out_shape=[jax.ShapeDtypeStruct((_NP, _H), f32),
                   jax.ShapeDtypeStruct((_NP, _H), f32)],
    )(xp, W_init, _rep8(b_init), W0, deg)

    gather_k = _sc_gather()

    def agg_layer(y):
        msg = gather_k(row_s3, y)
        return pl.pallas_call(
            _agg_body,
            grid_spec=pltpu.PrefetchScalarGridSpec(
                num_scalar_prefetch=1,
                grid=(_NB,),
                in_specs=[
                    pl.BlockSpec((1, 1, _EB), lambda b, nbr: (b, 0, 0)),
                    pl.BlockSpec((_EB, _H), lambda b, nbr: (b, 0)),
                ],
                out_specs=pl.BlockSpec((_NP, _H), lambda b, nbr: (0, 0)),
                scratch_shapes=[pltpu.VMEM((_NP, _H), f32)],
            ),
            out_shape=jax.ShapeDtypeStruct((_NP, _H), f32),
        )(nb, col_s3, msg)

    def mid(p, y_prev, b_prev, g, bbeta, w_next):
        return pl.pallas_call(
            _mid_body,
            grid=(_NP // _BLK,),
            in_specs=[
                _row_spec(_BLK), _row_spec(_BLK), _row_spec(_BLK),
                _full_spec((8, _H)), _full_spec((8, _H)),
                _full_spec((8, _H)), _full_spec((_H, _H)),
            ],
            out_specs=_row_spec(_BLK),
            out_shape=jax.ShapeDtypeStruct((_NP, _H), f32),
        )(p, y_prev, dinv, _rep8(b_prev), _rep8(g), _rep8(bbeta), w_next)

    agg0 = agg_layer(y0)
    y1 = mid(agg0, y0, b0, ln0_g, ln0_b, W1)
    agg1 = agg_layer(y1)
    y2 = mid(agg1, y1, b1, ln1_g, ln1_b, W2)
    agg2 = agg_layer(y2)

    wcls_p = jnp.zeros((_H, _H), f32).at[:, :10].set(W_cls)
    bcls_p = jnp.zeros((_H,), f32).at[:10].set(b_cls)
    batch3 = batch.astype(jnp.int32).reshape(_N // _FBLK, 1, _FBLK)

    emb, logp = pl.pallas_call(
        _final_body,
        grid=(_N // _FBLK,),
        in_specs=[
            _row_spec(_FBLK), _row_spec(_FBLK), _row_spec(_FBLK),
            _full_spec((8, _H)),
            pl.BlockSpec((1, 1, _FBLK), lambda i: (i, 0, 0)),
            _full_spec((_H, _H)), _full_spec((8, _H)),
        ],
        out_specs=[_full_spec((_H, _H)), _full_spec((_H, _H))],
        out_shape=[jax.ShapeDtypeStruct((_H, _H), f32),
                   jax.ShapeDtypeStruct((_H, _H), f32)],
        scratch_shapes=[pltpu.VMEM((_H, _H), f32),
                        pltpu.VMEM((_H, _H), f32)],
    )(agg2, y2, dinv, _rep8(b2), batch3, wcls_p, _rep8(bcls_p))

    return (emb[:_G, :10], logp[:_G, :10])


# speedup vs baseline: 4.4193x; 1.0181x over previous
import functools

import jax
import jax.numpy as jnp
from jax import lax
from jax.experimental import pallas as pl
from jax.experimental.pallas import tpu as pltpu
from jax.experimental.pallas import tpu_sc as plsc

_N = 10000
_E = 320000
_H = 128
_G = 100
_NW = 32
_CK = 128
_CHUNKS = 80
_EP = _NW * _CHUNKS * _CK
_NP = 10240
_EB = 1024
_NB = _EP // _EB
_W = 256
_SENT = _NP + 64


def _gather_body(row_hbm, y_hbm, out_hbm, row_v,
                 buf0, buf1, buf2, buf3,
                 g0, g1, g2, g3, w0, w1, w2, w3):
    cid = lax.axis_index("c")
    sid = lax.axis_index("s")
    wid = sid * 2 + cid
    pltpu.sync_copy(row_hbm.at[wid], row_v)
    bufs = (buf0, buf1, buf2, buf3)
    gs = (g0, g1, g2, g3)
    ws = (w0, w1, w2, w3)

    def _obase(j):
        return out_hbm.at[pl.ds((wid * _CHUNKS + j) * _CK, _CK)]

    n_quads = _CHUNKS // 4

    def _body(i, _):
        j0 = 4 * i
        for p in range(4):
            @pl.when(i > 0)
            def _():
                pltpu.make_async_copy(bufs[p], _obase(j0 + p - 4),
                                      ws[p]).wait()
            pltpu.async_copy(y_hbm.at[row_v.at[j0 + p]], bufs[p], gs[p])
        for p in range(4):
            pltpu.make_async_copy(y_hbm.at[row_v.at[j0 + p]], bufs[p],
                                  gs[p]).wait()
            pltpu.async_copy(bufs[p], _obase(j0 + p), ws[p])
        return 0

    lax.fori_loop(0, n_quads, _body, 0)
    for p in range(4):
        pltpu.make_async_copy(bufs[p], _obase(_CHUNKS - 4 + p), ws[p]).wait()


@functools.lru_cache(maxsize=1)
def _sc_gather():
    mesh = plsc.VectorSubcoreMesh(core_axis_name="c", subcore_axis_name="s")
    return pl.kernel(
        _gather_body,
        out_type=jax.ShapeDtypeStruct((_EP, _H), jnp.float32),
        mesh=mesh,
        scratch_types=(
            [pltpu.VMEM((_CHUNKS, _CK), jnp.int32)]
            + [pltpu.VMEM((_CK, _H), jnp.float32)] * 4
            + [pltpu.SemaphoreType.DMA] * 8
        ),
    )


def _deg_body(nb_ref, colb_ref, deg_ref, dacc_ref):
    b = pl.program_id(0)

    @pl.when(b == 0)
    def _():
        dacc_ref[...] = jnp.zeros((_NP, _H), jnp.float32)

    nb = nb_ref[b]
    lid = colb_ref[0, 0:1, :] - nb
    rows = lax.broadcasted_iota(jnp.int32, (_W, _EB), 0)
    oh = jnp.where(rows == jnp.broadcast_to(lid, (_W, _EB)), 1.0, 0.0)
    dsum = jnp.sum(oh, axis=1, keepdims=True)
    dacc_ref[pl.ds(nb, _W), :] += jnp.broadcast_to(dsum, (_W, _H))

    @pl.when(b == pl.num_programs(0) - 1)
    def _():
        deg_ref[...] = dacc_ref[...]


def _agg_body(nb_ref, colb_ref, msg_ref, agg_ref, acc_ref):
    b = pl.program_id(0)

    @pl.when(b == 0)
    def _():
        acc_ref[...] = jnp.zeros((_NP, _H), jnp.float32)

    nb = nb_ref[b]
    lid = colb_ref[0, 0:1, :] - nb
    rows = lax.broadcasted_iota(jnp.int32, (_W, _EB), 0)
    oh = jnp.where(rows == jnp.broadcast_to(lid, (_W, _EB)), 1.0, 0.0)
    acc_ref[pl.ds(nb, _W), :] += jnp.dot(
        oh, msg_ref[...], preferred_element_type=jnp.float32)

    @pl.when(b == pl.num_programs(0) - 1)
    def _():
        agg_ref[...] = acc_ref[...]


_BLK = 2048


def _pre_body(x_ref, w_init_ref, b_init_ref, w0_ref, deg_ref,
              y0_ref, dinv_ref):
    dinv = jax.lax.rsqrt(deg_ref[...] + 1.0)
    dinv_ref[...] = dinv
    x1 = jnp.dot(x_ref[...], w_init_ref[...],
                 preferred_element_type=jnp.float32) + b_init_ref[0:1, :]
    y0_ref[...] = jnp.dot(x1, w0_ref[...],
                          preferred_element_type=jnp.float32) * dinv


def _mid_body(p_ref, y_ref, dinv_ref, b_ref, g_ref, bb_ref, w_ref, yn_ref):
    dinv = dinv_ref[...]
    h = (p_ref[...] + y_ref[...]) * dinv + b_ref[0:1, :]
    h = jnp.maximum(h, 0.0)
    m = jnp.mean(h, axis=-1, keepdims=True)
    c = h - m
    v = jnp.mean(c * c, axis=-1, keepdims=True)
    h = c * jax.lax.rsqrt(v + 1e-5) * g_ref[0:1, :] + bb_ref[0:1, :]
    yn_ref[...] = jnp.dot(h, w_ref[...],
                          preferred_element_type=jnp.float32) * dinv


_FBLK = 1000


def _final_body(p_ref, y_ref, dinv_ref, b_ref, batch_ref,
                wcls_ref, bcls_ref, emb_ref, logp_ref, pool_ref, cnt_ref):
    pid = pl.program_id(0)

    @pl.when(pid == 0)
    def _():
        pool_ref[...] = jnp.zeros((_H, _H), jnp.float32)
        cnt_ref[...] = jnp.zeros((_H, _H), jnp.float32)

    x3 = (p_ref[...] + y_ref[...]) * dinv_ref[...] + b_ref[0:1, :]
    gids = lax.broadcasted_iota(jnp.int32, (_H, _FBLK), 0)
    bb = jnp.broadcast_to(batch_ref[0, 0:1, :], (_H, _FBLK))
    oh = jnp.where(gids == bb, 1.0, 0.0)
    pool_ref[...] += jnp.dot(oh, x3, preferred_element_type=jnp.float32)
    cnt_ref[...] += jnp.broadcast_to(
        jnp.sum(oh, axis=1, keepdims=True), (_H, _H))

    @pl.when(pid == pl.num_programs(0) - 1)
    def _():
        mean = pool_ref[...] / jnp.maximum(cnt_ref[...], 1.0)
        logits = jnp.dot(mean, wcls_ref[...],
                         preferred_element_type=jnp.float32) + bcls_ref[0:1, :]
        emb_ref[...] = logits
        cmask = lax.broadcasted_iota(jnp.int32, (_H, _H), 1) < 10
        ml = jnp.where(cmask, logits, -1e30)
        mx = jnp.max(ml, axis=-1, keepdims=True)
        lse = mx + jnp.log(jnp.sum(jnp.exp(ml - mx), axis=-1, keepdims=True))
        logp_ref[...] = logits - lse


def _row_spec(blk):
    return pl.BlockSpec((blk, _H), lambda i: (i, 0))


def _full_spec(shape):
    return pl.BlockSpec(shape, lambda i: tuple(0 for _ in shape))


def _rep8(v):
    return jnp.tile(v[None, :], (8, 1))


def kernel(x, edge_index, batch, ptr, centrality, W_init, b_init, W0, b0,
           W1, b1, W2, b2, ln0_g, ln0_b, ln1_g, ln1_b, W_cls, b_cls):
    f32 = jnp.float32
    pad_e = _EP - _E
    rowp = jnp.concatenate(
        [edge_index[0].astype(jnp.int32), jnp.zeros((pad_e,), jnp.int32)])
    colp = jnp.concatenate(
        [edge_index[1].astype(jnp.int32),
         jnp.full((pad_e,), _SENT, jnp.int32)])
    order = jnp.argsort(colp)
    row_s = rowp[order]
    col_s = colp[order]
    row_s3 = row_s.reshape(_NW, _CHUNKS, _CK)
    col_s3 = col_s.reshape(_NB, 1, _EB)
    nb = jnp.minimum((col_s[::_EB] // 8) * 8, _NP - _W).astype(jnp.int32)

    xp = jnp.zeros((_NP, _H), f32).at[:_N].set(x)

    deg = pl.pallas_call(
        _deg_body,
        grid_spec=pltpu.PrefetchScalarGridSpec(
            num_scalar_prefetch=1,
            grid=(_NB,),
            in_specs=[pl.BlockSpec((1, 1, _EB), lambda b, nbr: (b, 0, 0))],
            out_specs=pl.BlockSpec((_NP, _H), lambda b, nbr: (0, 0)),
            scratch_shapes=[pltpu.VMEM((_NP, _H), f32)],
        ),
        out_shape=jax.ShapeDtypeStruct((_NP, _H), f32),
    )(nb, col_s3)

    y0, dinv = pl.pallas_call(
        _pre_body,
        grid=(_NP // _BLK,),
        in_specs=[
            _row_spec(_BLK), _full_spec((_H, _H)), _full_spec((8, _H)),
            _full_spec((_H, _H)), _row_spec(_BLK),
        ],
        out_specs=[_row_spec(_BLK), _row_spec(_BLK)],
        out_shape=[jax.ShapeDtypeStruct((_NP, _H), f32),
                   jax.ShapeDtypeStruct((_NP, _H), f32)],
    )(xp, W_init, _rep8(b_init), W0, deg)

    gather_k = _sc_gather()

    def agg_layer(y):
        msg = gather_k(row_s3, y)
        return pl.pallas_call(
            _agg_body,
            grid_spec=pltpu.PrefetchScalarGridSpec(
                num_scalar_prefetch=1,
                grid=(_NB,),
                in_specs=[
                    pl.BlockSpec((1, 1, _EB), lambda b, nbr: (b, 0, 0)),
                    pl.BlockSpec((_EB, _H), lambda b, nbr: (b, 0)),
                ],
                out_specs=pl.BlockSpec((_NP, _H), lambda b, nbr: (0, 0)),
                scratch_shapes=[pltpu.VMEM((_NP, _H), f32)],
            ),
            out_shape=jax.ShapeDtypeStruct((_NP, _H), f32),
        )(nb, col_s3, msg)

    def mid(p, y_prev, b_prev, g, bbeta, w_next):
        return pl.pallas_call(
            _mid_body,
            grid=(_NP // _BLK,),
            in_specs=[
                _row_spec(_BLK), _row_spec(_BLK), _row_spec(_BLK),
                _full_spec((8, _H)), _full_spec((8, _H)),
                _full_spec((8, _H)), _full_spec((_H, _H)),
            ],
            out_specs=_row_spec(_BLK),
            out_shape=jax.ShapeDtypeStruct((_NP, _H), f32),
        )(p, y_prev, dinv, _rep8(b_prev), _rep8(g), _rep8(bbeta), w_next)

    agg0 = agg_layer(y0)
    y1 = mid(agg0, y0, b0, ln0_g, ln0_b, W1)
    agg1 = agg_layer(y1)
    y2 = mid(agg1, y1, b1, ln1_g, ln1_b, W2)
    agg2 = agg_layer(y2)

    wcls_p = jnp.zeros((_H, _H), f32).at[:, :10].set(W_cls)
    bcls_p = jnp.zeros((_H,), f32).at[:10].set(b_cls)
    batch3 = batch.astype(jnp.int32).reshape(_N // _FBLK, 1, _FBLK)

    emb, logp = pl.pallas_call(
        _final_body,
        grid=(_N // _FBLK,),
        in_specs=[
            _row_spec(_FBLK), _row_spec(_FBLK), _row_spec(_FBLK),
            _full_spec((8, _H)),
            pl.BlockSpec((1, 1, _FBLK), lambda i: (i, 0, 0)),
            _full_spec((_H, _H)), _full_spec((8, _H)),
        ],
        out_specs=[_full_spec((_H, _H)), _full_spec((_H, _H))],
        out_shape=[jax.ShapeDtypeStruct((_H, _H), f32),
                   jax.ShapeDtypeStruct((_H, _H), f32)],
        scratch_shapes=[pltpu.VMEM((_H, _H), f32),
                        pltpu.VMEM((_H, _H), f32)],
    )(agg2, y2, dinv, _rep8(b2), batch3, wcls_p, _rep8(bcls_p))

    return (emb[:_G, :10], logp[:_G, :10])
